# baseline (speedup 1.0000x reference)
import jax
import jax.numpy as jnp
from jax import lax
from jax.experimental import pallas as pl
from jax.experimental.pallas import tpu as pltpu
from jax.experimental.pallas import tpu_sc as plsc

D = 64
T = 256
B = 1024
N = B * T
NC = 2
NS = 16
NW = NC * NS
BPW = N // NW
C = 256
NCHUNK = BPW // C
L = 16


def _body(x_hbm, tok_hbm, pos_hbm, out_hbm, idx_v, rows0, rows1, pos_v,
          gsem, osem):
    wid = lax.axis_index("s") * NC + lax.axis_index("c")
    base = wid * BPW
    rows = (rows0, rows1)

    pltpu.sync_copy(x_hbm.at[pl.ds(base, BPW)], idx_v)
    pltpu.sync_copy(pos_hbm, pos_v)

    def start_gather(c, b):
        rv = rows[b]

        def blk_body(blk, _):
            vec = idx_v[pl.ds(c * C + blk * L, L)]
            for j in range(L):
                pltpu.async_copy(tok_hbm.at[vec[j]], rv.at[blk * L + j],
                                 gsem.at[b])
            return 0

        lax.fori_loop(0, C // L, blk_body, 0)

    def wait_gather(b):
        def blk_body(blk, _):
            for j in range(2):
                pltpu.make_async_copy(tok_hbm.at[pl.ds(0, 8)],
                                      rows[b].at[pl.ds(0, 8)],
                                      gsem.at[b]).wait()
            return 0

        lax.fori_loop(0, C // L, blk_body, 0)

    def add_and_gather(src_b, dst_b, cg, pred):
        rs, rd = rows[src_b], rows[dst_b]

        def blk_body(i, _):
            @pl.when(pred)
            def _():
                vec = idx_v[pl.ds(cg * C + i * L, L)]
                for j in range(L):
                    pltpu.async_copy(tok_hbm.at[vec[j]], rd.at[i * L + j],
                                     gsem.at[dst_b])
            for tt in range(L):
                t = i * L + tt
                for j in range(D // L):
                    rs[t, pl.ds(j * L, L)] += pos_v[t, pl.ds(j * L, L)]
            return 0

        lax.fori_loop(0, C // L, blk_body, 0)

    def out_start(c, b):
        pltpu.make_async_copy(rows[b], out_hbm.at[pl.ds(base + c * C, C)],
                              osem.at[b]).start()

    def out_wait(b):
        pltpu.make_async_copy(rows[b], out_hbm.at[pl.ds(base, C)],
                              osem.at[b]).wait()

    start_gather(0, 0)
    H = NCHUNK // 2

    def chunk_pair(k, _):
        c0 = 2 * k
        c1 = c0 + 1
        wait_gather(0)

        @pl.when(k > 0)
        def _():
            out_wait(1)
        add_and_gather(0, 1, c1, True)
        out_start(c0, 0)
        wait_gather(1)
        out_wait(0)
        add_and_gather(1, 0, jnp.minimum(c0 + 2, NCHUNK - 1), k < H - 1)
        out_start(c1, 1)
        return 0

    lax.fori_loop(0, H, chunk_pair, 0)
    out_wait(1)


def kernel(x, token_table, pos_table):
    xf = x.reshape(N).astype(jnp.int32)
    run = pl.kernel(
        _body,
        out_type=jax.ShapeDtypeStruct((N, D), jnp.float32),
        mesh=plsc.VectorSubcoreMesh(core_axis_name="c", subcore_axis_name="s"),
        compiler_params=pltpu.CompilerParams(use_tc_tiling_on_sc=True),
        scratch_types=[
            pltpu.VMEM((BPW,), jnp.int32),
            pltpu.VMEM((C, D), jnp.float32),
            pltpu.VMEM((C, D), jnp.float32),
            pltpu.VMEM((T, D), jnp.float32),
            pltpu.SemaphoreType.DMA((2,)),
            pltpu.SemaphoreType.DMA((2,)),
        ],
    )
    out = run(xf, token_table, pos_table)
    return out.reshape(B, T, D)


# speedup vs baseline: 1.0279x; 1.0279x over previous
import jax
import jax.numpy as jnp
from jax import lax
from jax.experimental import pallas as pl
from jax.experimental.pallas import tpu as pltpu
from jax.experimental.pallas import tpu_sc as plsc

D = 64
T = 256
B = 1024
N = B * T
NC = 2
NS = 16
NW = NC * NS
BPW = N // NW
C = 128
NCHUNK = BPW // C
L = 16
NBUF = 4


def _body(x_hbm, tok_hbm, pos_hbm, out_hbm, idx_v, r0, r1, r2, r3, pos_v,
          gsem, osem):
    wid = lax.axis_index("s") * NC + lax.axis_index("c")
    base = wid * BPW
    rows = (r0, r1, r2, r3)

    pltpu.sync_copy(x_hbm.at[pl.ds(base, BPW)], idx_v)
    pltpu.sync_copy(pos_hbm, pos_v)

    def start_gather(c, b):
        rv = rows[b]

        def blk_body(blk, _):
            vec = idx_v[pl.ds(c * C + blk * L, L)]
            for j in range(L):
                pltpu.async_copy(tok_hbm.at[vec[j]], rv.at[blk * L + j],
                                 gsem.at[b])
            return 0

        lax.fori_loop(0, C // L, blk_body, 0)

    def wait_gather(b):
        def blk_body(blk, _):
            for j in range(2):
                pltpu.make_async_copy(tok_hbm.at[pl.ds(0, 8)],
                                      rows[b].at[pl.ds(0, 8)],
                                      gsem.at[b]).wait()
            return 0

        lax.fori_loop(0, C // L, blk_body, 0)

    def add_and_gather(src_b, poff, cg, dst_b, pred):
        rs, rd = rows[src_b], rows[dst_b]

        def blk_body(i, _):
            @pl.when(pred)
            def _():
                vec = idx_v[pl.ds(cg * C + i * L, L)]
                for j in range(L):
                    pltpu.async_copy(tok_hbm.at[vec[j]], rd.at[i * L + j],
                                     gsem.at[dst_b])
            for tt in range(L):
                t = i * L + tt
                for j in range(D // L):
                    rs[t, pl.ds(j * L, L)] += pos_v[t + poff, pl.ds(j * L, L)]
            return 0

        lax.fori_loop(0, C // L, blk_body, 0)

    def out_start(c, b):
        pltpu.make_async_copy(rows[b], out_hbm.at[pl.ds(base + c * C, C)],
                              osem.at[b]).start()

    def out_wait(b):
        pltpu.make_async_copy(rows[b], out_hbm.at[pl.ds(base, C)],
                              osem.at[b]).wait()

    start_gather(0, 0)
    start_gather(1, 1)
    H = NCHUNK // NBUF
    true_ = jnp.bool_(True)

    def ring(k, _):
        c0 = NBUF * k
        for i in range(NBUF):
            c = c0 + i
            wait_gather(i)
            if i < 2:
                @pl.when(k > 0)
                def _():
                    out_wait(i + 2)
                add_and_gather(i, 128 * (i & 1), c + 2, i + 2, true_)
            else:
                out_wait(i - 2)
                add_and_gather(i, 128 * (i & 1), jnp.minimum(c + 2, NCHUNK - 1),
                               i - 2, k < H - 1)
            out_start(c, i)
        return 0

    lax.fori_loop(0, H, ring, 0)
    out_wait(2)
    out_wait(3)


def kernel(x, token_table, pos_table):
    xf = x.reshape(N).astype(jnp.int32)
    run = pl.kernel(
        _body,
        out_type=jax.ShapeDtypeStruct((N, D), jnp.float32),
        mesh=plsc.VectorSubcoreMesh(core_axis_name="c", subcore_axis_name="s"),
        compiler_params=pltpu.CompilerParams(use_tc_tiling_on_sc=True),
        scratch_types=[
            pltpu.VMEM((BPW,), jnp.int32),
            pltpu.VMEM((C, D), jnp.float32),
            pltpu.VMEM((C, D), jnp.float32),
            pltpu.VMEM((C, D), jnp.float32),
            pltpu.VMEM((C, D), jnp.float32),
            pltpu.VMEM((T, D), jnp.float32),
            pltpu.SemaphoreType.DMA((NBUF,)),
            pltpu.SemaphoreType.DMA((NBUF,)),
        ],
    )
    out = run(xf, token_table, pos_table)
    return out.reshape(B, T, D)


# speedup vs baseline: 1.0285x; 1.0006x over previous
import jax
import jax.numpy as jnp
from jax import lax
from jax.experimental import pallas as pl
from jax.experimental.pallas import tpu as pltpu
from jax.experimental.pallas import tpu_sc as plsc

D = 64
T = 256
B = 1024
N = B * T
NC = 2
NS = 16
NW = NC * NS
BPW = N // NW
C = 128
NCHUNK = BPW // C
L = 16
NBUF = 4


def _body(x_hbm, tok_hbm, pos_hbm, out_hbm, idx_v, r0, r1, r2, r3, pos_v,
          gsem, osem):
    wid = lax.axis_index("s") * NC + lax.axis_index("c")
    base = wid * BPW
    rows = (r0, r1, r2, r3)

    pltpu.sync_copy(x_hbm.at[pl.ds(base, BPW)], idx_v)
    pltpu.sync_copy(pos_hbm, pos_v)

    def start_gather(c, b):
        rv = rows[b]

        def blk_body(blk, _):
            vec = idx_v[pl.ds(c * C + blk * L, L)]
            for j in range(L):
                pltpu.async_copy(tok_hbm.at[vec[j]], rv.at[blk * L + j],
                                 gsem.at[b])
            return 0

        lax.fori_loop(0, C // L, blk_body, 0)

    def wait_gather(b):
        def blk_body(blk, _):
            for j in range(2):
                pltpu.make_async_copy(tok_hbm.at[pl.ds(0, 8)],
                                      rows[b].at[pl.ds(0, 8)],
                                      gsem.at[b]).wait()
            return 0

        lax.fori_loop(0, C // L, blk_body, 0)

    def add_and_gather(src_b, poff, cg, dst_b, pred):
        rs, rd = rows[src_b], rows[dst_b]

        def blk_body(i, _):
            @pl.when(pred)
            def _():
                vec = idx_v[pl.ds(cg * C + i * L, L)]
                for j in range(L):
                    pltpu.async_copy(tok_hbm.at[vec[j]], rd.at[i * L + j],
                                     gsem.at[dst_b])
            for tt in range(L):
                t = i * L + tt
                for j in range(D // L):
                    plsc.addupdate(rs.at[t, pl.ds(j * L, L)],
                                   pos_v[t + poff, pl.ds(j * L, L)])
            return 0

        lax.fori_loop(0, C // L, blk_body, 0)

    def out_start(c, b):
        pltpu.make_async_copy(rows[b], out_hbm.at[pl.ds(base + c * C, C)],
                              osem.at[b]).start()

    def out_wait(b):
        pltpu.make_async_copy(rows[b], out_hbm.at[pl.ds(base, C)],
                              osem.at[b]).wait()

    start_gather(0, 0)
    start_gather(1, 1)
    H = NCHUNK // NBUF
    true_ = jnp.bool_(True)

    def ring(k, _):
        c0 = NBUF * k
        for i in range(NBUF):
            c = c0 + i
            wait_gather(i)
            if i < 2:
                @pl.when(k > 0)
                def _():
                    out_wait(i + 2)
                add_and_gather(i, 128 * (i & 1), c + 2, i + 2, true_)
            else:
                out_wait(i - 2)
                add_and_gather(i, 128 * (i & 1), jnp.minimum(c + 2, NCHUNK - 1),
                               i - 2, k < H - 1)
            out_start(c, i)
        return 0

    lax.fori_loop(0, H, ring, 0)
    out_wait(2)
    out_wait(3)


def kernel(x, token_table, pos_table):
    xf = x.reshape(N).astype(jnp.int32)
    run = pl.kernel(
        _body,
        out_type=jax.ShapeDtypeStruct((N, D), jnp.float32),
        mesh=plsc.VectorSubcoreMesh(core_axis_name="c", subcore_axis_name="s"),
        compiler_params=pltpu.CompilerParams(use_tc_tiling_on_sc=True),
        scratch_types=[
            pltpu.VMEM((BPW,), jnp.int32),
            pltpu.VMEM((C, D), jnp.float32),
            pltpu.VMEM((C, D), jnp.float32),
            pltpu.VMEM((C, D), jnp.float32),
            pltpu.VMEM((C, D), jnp.float32),
            pltpu.VMEM((T, D), jnp.float32),
            pltpu.SemaphoreType.DMA((NBUF,)),
            pltpu.SemaphoreType.DMA((NBUF,)),
        ],
    )
    out = run(xf, token_table, pos_table)
    return out.reshape(B, T, D)
